# baseline (speedup 1.0000x reference)
import functools

import jax
import jax.numpy as jnp
from jax import lax
from jax.experimental import pallas as pl
from jax.experimental.pallas import tpu as pltpu
from jax.experimental.pallas import tpu_sc as plsc

VOCAB = 1_000_000
D = 64
BATCH = 4096
HIST = 200

NC = 2
NS = 16
NW = NC * NS
B = BATCH * HIST
B_PER_W = B // NW
CHUNK = 128
NSTEP = B_PER_W // CHUNK
NBUF = 4


def _emb_body(tok_hbm, w_hbm, out_hbm, idx_v, rows_v, gsem, ssem):
    wid = lax.axis_index("s") * NC + lax.axis_index("c")
    base = wid * B_PER_W

    pltpu.sync_copy(tok_hbm.at[wid], idx_v)

    def fire_gather(i, b):
        pltpu.async_copy(w_hbm.at[idx_v.at[i]], rows_v.at[b], gsem)

    def wait_gather(i, b):
        pltpu.make_async_copy(w_hbm.at[idx_v.at[i]], rows_v.at[b], gsem).wait()

    def fire_scatter(i, b):
        pltpu.async_copy(
            rows_v.at[b], out_hbm.at[pl.ds(base + i * CHUNK, CHUNK)], ssem)

    def wait_scatter(i, b):
        pltpu.make_async_copy(
            rows_v.at[b], out_hbm.at[pl.ds(base + i * CHUNK, CHUNK)], ssem
        ).wait()

    for b in range(NBUF):
        fire_gather(b, b)

    @pl.loop(0, (NSTEP - NBUF) // NBUF)
    def _steady(g):
        for b in range(NBUF):
            i = g * NBUF + b
            wait_gather(i, b)
            fire_scatter(i, b)
            wait_scatter(i, b)
            fire_gather(i + NBUF, b)

    for b in range(NBUF):
        i = NSTEP - NBUF + b
        wait_gather(i, b)
        fire_scatter(i, b)
        wait_scatter(i, b)


@jax.jit
def _embed(tok3, weight):
    mesh = plsc.VectorSubcoreMesh(core_axis_name="c", subcore_axis_name="s")
    run = pl.kernel(
        _emb_body,
        out_type=jax.ShapeDtypeStruct((B, D), jnp.float32),
        mesh=mesh,
        scratch_types=[
            pltpu.VMEM((NSTEP, CHUNK), jnp.int32),
            pltpu.VMEM((NBUF, CHUNK, D), jnp.float32),
            pltpu.SemaphoreType.DMA,
            pltpu.SemaphoreType.DMA,
        ],
        compiler_params=pltpu.CompilerParams(use_tc_tiling_on_sc=False),
    )
    return run(tok3, weight)


def kernel(tokens, weight):
    tok3 = tokens.reshape(NW, NSTEP, CHUNK).astype(jnp.int32)
    out = _embed(tok3, weight)
    return out.reshape(BATCH, HIST, D)


# speedup vs baseline: 1.0012x; 1.0012x over previous
import jax
import jax.numpy as jnp
from jax import lax
from jax.experimental import pallas as pl
from jax.experimental.pallas import tpu as pltpu
from jax.experimental.pallas import tpu_sc as plsc

VOCAB = 1_000_000
D = 64
BATCH = 4096
HIST = 200

NC = 2
NS = 16
NW = NC * NS
ROWS_PER_W = BATCH // NW
CA = 128
CB = HIST - CA
NBUF = 4
NSTEP = ROWS_PER_W * 2


def _emb_body(tok_hbm, w_hbm, out_hbm, idx_v, rows_v, gsem, ssem):
    wid = lax.axis_index("s") * NC + lax.axis_index("c")
    row0 = wid * ROWS_PER_W

    pltpu.sync_copy(tok_hbm.at[pl.ds(row0, ROWS_PER_W)], idx_v)

    def gather_desc(r, b):
        half = b & 1
        off, n = (0, CA) if half == 0 else (CA, CB)
        return pltpu.make_async_copy(
            w_hbm.at[idx_v.at[r, pl.ds(off, n)]],
            rows_v.at[b, pl.ds(0, n)],
            gsem,
        )

    def scatter_desc(r, b):
        half = b & 1
        off, n = (0, CA) if half == 0 else (CA, CB)
        return pltpu.make_async_copy(
            rows_v.at[b, pl.ds(0, n)],
            out_hbm.at[row0 + r, pl.ds(off, n)],
            ssem,
        )

    for b in range(NBUF):
        gather_desc(b >> 1, b).start()

    @pl.loop(0, (NSTEP - NBUF) // NBUF)
    def _steady(g):
        for b in range(NBUF):
            r = 2 * g + (b >> 1)
            gather_desc(r, b).wait()
            scatter_desc(r, b).start()
            scatter_desc(r, b).wait()
            gather_desc(r + 2, b).start()

    for b in range(NBUF):
        r = ROWS_PER_W - 2 + (b >> 1)
        gather_desc(r, b).wait()
        scatter_desc(r, b).start()
        scatter_desc(r, b).wait()


def kernel(tokens, weight):
    if tokens.dtype != jnp.int32:
        tokens = tokens.astype(jnp.int32)
    mesh = plsc.VectorSubcoreMesh(core_axis_name="c", subcore_axis_name="s")
    run = pl.kernel(
        _emb_body,
        out_type=jax.ShapeDtypeStruct((BATCH, HIST, D), jnp.float32),
        mesh=mesh,
        scratch_types=[
            pltpu.VMEM((ROWS_PER_W, HIST), jnp.int32),
            pltpu.VMEM((NBUF, CA, D), jnp.float32),
            pltpu.SemaphoreType.DMA,
            pltpu.SemaphoreType.DMA,
        ],
        compiler_params=pltpu.CompilerParams(use_tc_tiling_on_sc=False),
    )
    return run(tokens, weight)


# speedup vs baseline: 1.2270x; 1.2255x over previous
import jax
import jax.numpy as jnp
from jax import lax
from jax.experimental import pallas as pl
from jax.experimental.pallas import tpu as pltpu
from jax.experimental.pallas import tpu_sc as plsc

VOCAB = 1_000_000
D = 64
DP = 128
BATCH = 4096
HIST = 200

NC = 2
NS = 16
NW = NC * NS
B = BATCH * HIST
B_PER_W = B // NW
CHUNK = 128
NSTEP = B_PER_W // CHUNK
NBUF = 4


def _emb_body(tok_hbm, w_hbm, out_hbm, idx_v, rows_v, gsem, ssem):
    wid = lax.axis_index("s") * NC + lax.axis_index("c")
    base = wid * B_PER_W

    pltpu.sync_copy(tok_hbm.at[pl.ds(base, B_PER_W)], idx_v)

    def gather_desc(i, b):
        return pltpu.make_async_copy(
            w_hbm.at[idx_v.at[pl.ds(i * CHUNK, CHUNK)]], rows_v.at[b], gsem)

    def scatter_desc(i, b):
        return pltpu.make_async_copy(
            rows_v.at[b], out_hbm.at[pl.ds(base + i * CHUNK, CHUNK)], ssem)

    for b in range(NBUF):
        gather_desc(b, b).start()

    @pl.loop(0, (NSTEP - NBUF) // NBUF)
    def _steady(g):
        for b in range(NBUF):
            i = g * NBUF + b
            gather_desc(i, b).wait()
            scatter_desc(i, b).start()
            scatter_desc(i, b).wait()
            gather_desc(i + NBUF, b).start()

    for b in range(NBUF):
        i = NSTEP - NBUF + b
        gather_desc(i, b).wait()
        scatter_desc(i, b).start()
        scatter_desc(i, b).wait()


def kernel(tokens, weight):
    if tokens.dtype != jnp.int32:
        tokens = tokens.astype(jnp.int32)
    tf = tokens.reshape(B)
    wp = jnp.pad(weight, ((0, 0), (0, DP - D)))
    mesh = plsc.VectorSubcoreMesh(core_axis_name="c", subcore_axis_name="s")
    run = pl.kernel(
        _emb_body,
        out_type=jax.ShapeDtypeStruct((B, DP), jnp.float32),
        mesh=mesh,
        scratch_types=[
            pltpu.VMEM((B_PER_W,), jnp.int32),
            pltpu.VMEM((NBUF, CHUNK, DP), jnp.float32),
            pltpu.SemaphoreType.DMA,
            pltpu.SemaphoreType.DMA,
        ],
        compiler_params=pltpu.CompilerParams(use_tc_tiling_on_sc=False),
    )
    outp = run(tf, wp)
    return outp[:, :D].reshape(BATCH, HIST, D)


# speedup vs baseline: 1.3319x; 1.0855x over previous
import jax
import jax.numpy as jnp
from jax import lax
from jax.experimental import pallas as pl
from jax.experimental.pallas import tpu as pltpu
from jax.experimental.pallas import tpu_sc as plsc

VOCAB = 1_000_000
D = 64
DP = 128
BATCH = 4096
HIST = 200

NC = 2
NS = 16
NW = NC * NS
B = BATCH * HIST
B_PER_W = B // NW
CHUNK = 128
NSTEP = B_PER_W // CHUNK
NBUF = 4


def _emb_body(tok_hbm, w_hbm, out_hbm, idx_v, rows_v, gsem, ssem):
    wid = lax.axis_index("s") * NC + lax.axis_index("c")
    base = wid * B_PER_W

    pltpu.sync_copy(tok_hbm.at[pl.ds(base, B_PER_W)], idx_v)

    def gather_desc(i, b):
        return pltpu.make_async_copy(
            w_hbm.at[idx_v.at[pl.ds(i * CHUNK, CHUNK)]], rows_v.at[b], gsem)

    def scatter_desc(i, b):
        return pltpu.make_async_copy(
            rows_v.at[b],
            out_hbm.at[pl.ds(base + i * CHUNK, CHUNK), pl.ds(0, D)],
            ssem,
        )

    for b in range(NBUF):
        gather_desc(b, b).start()

    @pl.loop(0, (NSTEP - NBUF) // NBUF)
    def _steady(g):
        for b in range(NBUF):
            i = g * NBUF + b
            gather_desc(i, b).wait()
            scatter_desc(i, b).start()
            scatter_desc(i, b).wait()
            gather_desc(i + NBUF, b).start()

    for b in range(NBUF):
        i = NSTEP - NBUF + b
        gather_desc(i, b).wait()
        scatter_desc(i, b).start()
        scatter_desc(i, b).wait()


def kernel(tokens, weight):
    if tokens.dtype != jnp.int32:
        tokens = tokens.astype(jnp.int32)
    tf2 = tokens.reshape(B)
    mesh = plsc.VectorSubcoreMesh(core_axis_name="c", subcore_axis_name="s")
    run = pl.kernel(
        _emb_body,
        out_type=jax.ShapeDtypeStruct((B, DP), jnp.float32),
        mesh=mesh,
        scratch_types=[
            pltpu.VMEM((B_PER_W,), jnp.int32),
            pltpu.VMEM((NBUF, CHUNK, D), jnp.float32),
            pltpu.SemaphoreType.DMA,
            pltpu.SemaphoreType.DMA,
        ],
        compiler_params=pltpu.CompilerParams(use_tc_tiling_on_sc=False),
    )
    outp = run(tf2, weight)
    return outp[:, :D].reshape(BATCH, HIST, D)


# speedup vs baseline: 1.4323x; 1.0754x over previous
import jax
import jax.numpy as jnp
from jax import lax
from jax.experimental import pallas as pl
from jax.experimental.pallas import tpu as pltpu
from jax.experimental.pallas import tpu_sc as plsc

VOCAB = 1_000_000
D = 64
DP = 128
BATCH = 4096
HIST = 200

NC = 2
NS = 16
NW = NC * NS
B = BATCH * HIST
B_PER_W = B // NW
CHUNK = 128
NSTEP = B_PER_W // CHUNK
NBUF = 4


def _emb_body(tok_hbm, w_hbm, out_hbm, idx_v, rows_v, gsem, ssem):
    wid = lax.axis_index("s") * NC + lax.axis_index("c")
    base = wid * B_PER_W

    pltpu.sync_copy(tok_hbm.at[pl.ds(base, B_PER_W)], idx_v)

    def gather_desc(i, b):
        return pltpu.make_async_copy(
            w_hbm.at[idx_v.at[pl.ds(i * CHUNK, CHUNK)]], rows_v.at[b], gsem)

    def scatter_desc(i, b):
        return pltpu.make_async_copy(
            rows_v.at[b],
            out_hbm.at[pl.ds(base + i * CHUNK, CHUNK), pl.ds(0, D)],
            ssem,
        )

    for b in range(NBUF):
        gather_desc(b, b).start()

    @pl.loop(0, (NSTEP - NBUF) // NBUF)
    def _steady(g):
        for b in range(NBUF):
            i = g * NBUF + b
            gather_desc(i, b).wait()
            scatter_desc(i, b).start()
            scatter_desc(i, b).wait()
            gather_desc(i + NBUF, b).start()

    for b in range(NBUF):
        i = NSTEP - NBUF + b
        gather_desc(i, b).wait()
        scatter_desc(i, b).start()
        scatter_desc(i, b).wait()


def kernel(tokens, weight):
    if tokens.dtype != jnp.int32:
        tokens = tokens.astype(jnp.int32)
    wp = jnp.concatenate(
        [weight, jnp.zeros((VOCAB, DP - D), jnp.float32)], axis=1)
    w2 = wp.reshape(2 * VOCAB, D)
    tf2 = tokens.reshape(B) * 2
    mesh = plsc.VectorSubcoreMesh(core_axis_name="c", subcore_axis_name="s")
    run = pl.kernel(
        _emb_body,
        out_type=jax.ShapeDtypeStruct((B, DP), jnp.float32),
        mesh=mesh,
        scratch_types=[
            pltpu.VMEM((B_PER_W,), jnp.int32),
            pltpu.VMEM((NBUF, CHUNK, D), jnp.float32),
            pltpu.SemaphoreType.DMA,
            pltpu.SemaphoreType.DMA,
        ],
        compiler_params=pltpu.CompilerParams(use_tc_tiling_on_sc=False),
    )
    outp = run(tf2, w2)
    return outp[:, :D].reshape(BATCH, HIST, D)
